# baseline (speedup 1.0000x reference)
import functools

import jax
import jax.numpy as jnp
from jax.experimental import pallas as pl
from jax.experimental.pallas import tpu as pltpu
from jax.experimental.pallas import tpu_sc as plsc

B = 64
DIM = 768
HID = 3072
E = 8
HW = 24 * 24

PBAT = 2
NP = B // PBAT
NA = B // PBAT


def _pool_gate_kernel(x_ref, gamma_ref, beta_ref, wg_ref, bg_ref,
                      xn_ref, coef_ref, lt_ref, xp_ref):
    i = pl.program_id(0)

    @pl.when(i < NP)
    def _pool():
        v = x_ref[...].reshape(PBAT, HW, DIM)
        xp_ref[i] = jnp.sum(v, axis=1) * (1.0 / HW)

    @pl.when(i == NP)
    def _gating():
        xp = xp_ref[...].reshape(B, DIM)
        mu = jnp.mean(xp, axis=-1, keepdims=True)
        var = jnp.mean((xp - mu) ** 2, axis=-1, keepdims=True)
        xn = (xp - mu) * jax.lax.rsqrt(var + 1e-5) * gamma_ref[...] + beta_ref[...]
        xn_ref[...] = xn
        logits = jax.lax.dot_general(
            xn, wg_ref[...], (((1,), (1,)), ((), ())),
            preferred_element_type=jnp.float32,
            precision=jax.lax.Precision.HIGHEST) + bg_ref[...]
        lt_ref[...] = logits.T
        io = jax.lax.broadcasted_iota(jnp.int32, (B, E), 1)
        v1 = jnp.max(logits, axis=-1, keepdims=True)
        idx1 = jnp.min(jnp.where(logits == v1, io, E), axis=-1, keepdims=True)
        m1 = io == idx1
        logits_m = jnp.where(m1, -jnp.inf, logits)
        v2 = jnp.max(logits_m, axis=-1, keepdims=True)
        idx2 = jnp.min(jnp.where(logits_m == v2, io, E), axis=-1, keepdims=True)
        m2 = io == idx2
        z = jnp.exp(v2 - v1)
        w_a = 1.0 / (1.0 + z)
        w_b = z / (1.0 + z)
        coef_ref[...] = w_a * m1.astype(jnp.float32) + w_b * m2.astype(jnp.float32)


def _ffn_add_kernel(x_ref, xn_in_ref, coef_in_ref,
                    w1_ref, b1_ref, w2_ref, b2_ref,
                    o_ref, ffn_ref):
    i = pl.program_id(0)

    @pl.when(i == 0)
    def _init():
        ffn_ref[...] = jnp.zeros_like(ffn_ref)

    @pl.when(i < E)
    def _expert():
        e = i
        xn = xn_in_ref[...].astype(jnp.bfloat16)
        h = jax.lax.dot_general(
            xn, w1_ref[0].astype(jnp.bfloat16), (((1,), (0,)), ((), ())),
            preferred_element_type=jnp.float32) + b1_ref[0]
        h = h * jax.nn.sigmoid(h)
        o = jax.lax.dot_general(
            h.astype(jnp.bfloat16), w2_ref[0].astype(jnp.bfloat16),
            (((1,), (0,)), ((), ())),
            preferred_element_type=jnp.float32) + b2_ref[0]
        io = jax.lax.broadcasted_iota(jnp.int32, (B, E), 1)
        c = jnp.sum(jnp.where(io == e, coef_in_ref[...], 0.0), axis=-1,
                    keepdims=True)
        ffn_ref[...] += (c * o).reshape(NA, PBAT, DIM)

    @pl.when(i >= E)
    def _add():
        f = ffn_ref[i - E]
        v = x_ref[...].reshape(PBAT, HW, DIM)
        o_ref[...] = (v + f[:, None, :]).reshape(PBAT * HW, DIM)


def _sc_aux(logits_t):
    mesh = plsc.VectorSubcoreMesh(core_axis_name="c", subcore_axis_name="s")

    @functools.partial(
        pl.kernel,
        out_type=jax.ShapeDtypeStruct((16,), jnp.float32),
        mesh=mesh,
    )
    def aux_kernel(lt_hbm, out_hbm):
        def body(lt_vmem, out_vmem):
            nchunks = B // 16
            imp = jnp.zeros((16,), jnp.float32)
            load = jnp.zeros((16,), jnp.float32)
            lane = jax.lax.iota(jnp.int32, 16)

            gdn = jax.lax.GatherDimensionNumbers(
                offset_dims=(), collapsed_slice_dims=(0,),
                start_index_map=(0,))

            def allsum(v):
                for sh in (8, 4, 2, 1):
                    v = v + jax.lax.gather(
                        v, (lane ^ sh)[:, None], gdn, slice_sizes=(1,),
                        mode=jax.lax.GatherScatterMode.PROMISE_IN_BOUNDS)
                return v
            for c in range(nchunks):
                sl = pl.ds(c * 16, 16)
                rows = [lt_vmem[e, sl] for e in range(E)]
                m1 = rows[0]
                for e in range(1, E):
                    m1 = jnp.maximum(m1, rows[e])
                idx1 = jnp.full((16,), E, jnp.int32)
                for e in range(E - 1, -1, -1):
                    idx1 = jnp.where(rows[e] == m1, e, idx1)
                m2 = jnp.full((16,), -jnp.inf, jnp.float32)
                for e in range(E):
                    m2 = jnp.maximum(m2, jnp.where(idx1 == e, -jnp.inf, rows[e]))
                idx2 = jnp.full((16,), E, jnp.int32)
                for e in range(E - 1, -1, -1):
                    keep = (rows[e] == m2) & (idx1 != e)
                    idx2 = jnp.where(keep, e, idx2)
                pr = [jnp.exp(rows[e] - m1) for e in range(E)]
                den = pr[0]
                for e in range(1, E):
                    den = den + pr[e]
                for e in range(E):
                    sel = jnp.where(lane == e, 1.0, 0.0)
                    pe = allsum(pr[e] / den)
                    ce = allsum(jnp.where(idx1 == e, 1.0, 0.0)
                                + jnp.where(idx2 == e, 1.0, 0.0))
                    imp = imp + sel * pe
                    load = load + sel * ce
            emask = jnp.where(lane < E, 1.0, 0.0)

            def cv2(v):
                mean = allsum(v) * (1.0 / E)
                dv = (v - mean) * emask
                var = allsum(dv * dv) * (1.0 / E)
                return var / (mean * mean + 1e-10)

            aux = cv2(imp) + cv2(load)
            out_vmem[...] = jnp.where(lane == 0, aux, 0.0)

        pltpu.emit_pipeline(
            body,
            grid=(1,),
            in_specs=[pl.BlockSpec((E, B), lambda i: (0, 0))],
            out_specs=[pl.BlockSpec((16,), lambda i: (0,))],
            core_axis_name=("c", "s"),
            dimension_semantics=(pltpu.PARALLEL,),
        )(lt_hbm, out_hbm)

    return aux_kernel(logits_t)


@functools.partial(jax.jit, static_argnames=("interpret",))
def kernel(x, gamma, beta, Wg, bg, W1, b1, W2, b2, interpret=False):
    x2 = x.transpose(0, 2, 3, 1).reshape(B * HW, DIM)

    xn, coef, lt = pl.pallas_call(
        _pool_gate_kernel,
        grid=(NP + 1,),
        in_specs=[
            pl.BlockSpec((PBAT * HW, DIM),
                         lambda i: (jnp.minimum(i, NP - 1), 0)),
            pl.BlockSpec((1, DIM), lambda i: (0, 0)),
            pl.BlockSpec((1, DIM), lambda i: (0, 0)),
            pl.BlockSpec((E, DIM), lambda i: (0, 0)),
            pl.BlockSpec((1, E), lambda i: (0, 0)),
        ],
        out_specs=[
            pl.BlockSpec((B, DIM), lambda i: (0, 0)),
            pl.BlockSpec((B, E), lambda i: (0, 0)),
            pl.BlockSpec((E, B), lambda i: (0, 0)),
        ],
        out_shape=[
            jax.ShapeDtypeStruct((B, DIM), jnp.float32),
            jax.ShapeDtypeStruct((B, E), jnp.float32),
            jax.ShapeDtypeStruct((E, B), jnp.float32),
        ],
        scratch_shapes=[
            pltpu.VMEM((NP, PBAT, DIM), jnp.float32),
        ],
        interpret=interpret,
    )(x2, gamma.reshape(1, DIM), beta.reshape(1, DIM), Wg, bg.reshape(1, E))

    aux16 = _sc_aux(lt)

    def im_w(i):
        return (jnp.clip(i, 0, E - 1), 0, 0)

    def im_x(i):
        return (jnp.where(i < E, 0, i - E), 0)

    out = pl.pallas_call(
        _ffn_add_kernel,
        grid=(E + NA,),
        in_specs=[
            pl.BlockSpec((PBAT * HW, DIM), im_x),
            pl.BlockSpec((B, DIM), lambda i: (0, 0)),
            pl.BlockSpec((B, E), lambda i: (0, 0)),
            pl.BlockSpec((1, DIM, HID), im_w),
            pl.BlockSpec((1, 1, HID), im_w),
            pl.BlockSpec((1, HID, DIM), im_w),
            pl.BlockSpec((1, 1, DIM), im_w),
        ],
        out_specs=pl.BlockSpec((PBAT * HW, DIM), im_x),
        out_shape=jax.ShapeDtypeStruct((B * HW, DIM), jnp.float32),
        scratch_shapes=[
            pltpu.VMEM((NA, PBAT, DIM), jnp.float32),
        ],
        interpret=interpret,
    )(x2, xn, coef, W1, b1.reshape(E, 1, HID), W2, b2.reshape(E, 1, DIM))

    out4 = out.reshape(B, 24, 24, DIM).transpose(0, 3, 1, 2)
    return out4, aux16[0]
